# Initial kernel scaffold; baseline (speedup 1.0000x reference)
import jax
import jax.numpy as jnp
from jax.experimental import pallas as pl


def kernel(inputs, Wg, bg, W1, b1, W2, b2):
    raise NotImplementedError("write your pallas kernel here")


# speedup vs baseline: 2.2385x; 2.2385x over previous
import functools

import jax
import jax.numpy as jnp
from jax.experimental import pallas as pl
from jax.experimental.pallas import tpu as pltpu

_INTERPRET = False

_E = 8
_K = 2
_TM = 256


def _erf(x):
    return jax.lax.erf(x)


def _gelu(x):
    return 0.5 * x * (1.0 + _erf(x * 0.7071067811865476))


def _router_body(x_ref, wg_ref, bg_ref, out_ref):
    x = x_ref[...]
    wg = wg_ref[...]
    logits = jnp.dot(x, wg, preferred_element_type=jnp.float32) + bg_ref[...][None, :]
    m = jnp.max(logits, axis=-1, keepdims=True)
    p = jnp.exp(logits - m)
    g = p / jnp.sum(p, axis=-1, keepdims=True)
    lane = jax.lax.broadcasted_iota(jnp.int32, g.shape, 1)
    v0 = jnp.max(g, axis=-1, keepdims=True)
    e0 = jnp.min(jnp.where(g == v0, lane, _E), axis=-1, keepdims=True)
    g2 = jnp.where(lane == e0, -1.0, g)
    v1 = jnp.max(g2, axis=-1, keepdims=True)
    e1 = jnp.min(jnp.where(g2 == v1, lane, _E), axis=-1, keepdims=True)
    s = v0 + v1 + 1e-9
    keep = (lane == e0) | (lane == e1)
    out_ref[...] = jnp.where(keep, g / s, 0.0)


def _router(x2d, Wg, bg):
    T, D = x2d.shape
    TMR = 512
    return pl.pallas_call(
        _router_body,
        grid=(T // TMR,),
        in_specs=[
            pl.BlockSpec((TMR, D), lambda i: (i, 0)),
            pl.BlockSpec((D, _E), lambda i: (0, 0)),
            pl.BlockSpec((_E,), lambda i: (0,)),
        ],
        out_specs=pl.BlockSpec((TMR, _E), lambda i: (i, 0)),
        out_shape=jax.ShapeDtypeStruct((T, _E), jnp.float32),
        interpret=_INTERPRET,
    )(x2d, Wg, bg)


def _gmm_body(te_ref, xs_ref, w1_ref, b1_ref, w2_ref, b2_ref, rw_ref, out_ref):
    x = xs_ref[...].astype(jnp.bfloat16)
    h = jnp.dot(x, w1_ref[0], preferred_element_type=jnp.float32) + b1_ref[0, 0][None, :]
    h = _gelu(h)
    y = jnp.dot(h.astype(jnp.bfloat16), w2_ref[0], preferred_element_type=jnp.float32)
    y = _gelu(y + b2_ref[0, 0][None, :])
    out_ref[...] = y * rw_ref[0, 0][:, None]


def _gmm(tile_expert, xs, W1b, b1, W2b, b2, row_w3):
    CAP, D = xs.shape
    U1 = W1b.shape[2]
    U2 = W2b.shape[2]
    NT = CAP // _TM
    grid_spec = pltpu.PrefetchScalarGridSpec(
        num_scalar_prefetch=1,
        grid=(NT,),
        in_specs=[
            pl.BlockSpec((_TM, D), lambda i, te: (i, 0)),
            pl.BlockSpec((1, D, U1), lambda i, te: (te[i], 0, 0)),
            pl.BlockSpec((1, 1, U1), lambda i, te: (te[i], 0, 0)),
            pl.BlockSpec((1, U1, U2), lambda i, te: (te[i], 0, 0)),
            pl.BlockSpec((1, 1, U2), lambda i, te: (te[i], 0, 0)),
            pl.BlockSpec((1, 1, _TM), lambda i, te: (i, 0, 0)),
        ],
        out_specs=pl.BlockSpec((_TM, U2), lambda i, te: (i, 0)),
    )
    return pl.pallas_call(
        _gmm_body,
        grid_spec=grid_spec,
        out_shape=jax.ShapeDtypeStruct((CAP, U2), jnp.float32),
        interpret=_INTERPRET,
    )(tile_expert, xs, W1b, b1, W2b, b2, row_w3)


def kernel(inputs, Wg, bg, W1, b1, W2, b2):
    B, S, D = inputs.shape
    T = B * S
    U1 = W1.shape[2]
    U2 = W2.shape[2]
    x2d = inputs.reshape(T, D)

    gates = _router(x2d, Wg, bg)

    lane = jnp.arange(_E, dtype=jnp.int32)
    v0 = jnp.max(gates, axis=1, keepdims=True)
    e0 = jnp.min(jnp.where(gates == v0, lane[None, :], _E), axis=1)
    g2 = jnp.where(lane[None, :] == e0[:, None], -1.0, gates)
    v1 = jnp.max(g2, axis=1, keepdims=True)
    e1 = jnp.min(jnp.where(g2 == v1, lane[None, :], _E), axis=1)
    expert_flat = jnp.stack([e0, e1], axis=1).reshape(-1)
    w_flat = jnp.stack([v0[:, 0], v1[:, 0]], axis=1).reshape(-1)

    match = (expert_flat[:, None] == lane[None, :]).astype(jnp.int32)
    ranks_all = jnp.cumsum(match, axis=0) - 1
    rank = jnp.take_along_axis(ranks_all, expert_flat[:, None], axis=1)[:, 0]
    counts = jnp.sum(match, axis=0)
    padded = ((counts + _TM - 1) // _TM) * _TM
    offs = jnp.concatenate([jnp.zeros((1,), jnp.int32),
                            jnp.cumsum(padded)[:-1].astype(jnp.int32)])
    pos = offs[expert_flat] + rank

    CAP = _K * T + _E * _TM
    NT = CAP // _TM
    ends = (offs + padded) // _TM
    tiles = jnp.arange(NT, dtype=jnp.int32)
    tile_expert = jnp.minimum(
        jnp.sum(tiles[:, None] >= ends[None, :], axis=1), _E - 1
    ).astype(jnp.int32)

    row_token = jnp.zeros((CAP,), jnp.int32).at[pos].set(
        jnp.arange(_K * T, dtype=jnp.int32) // _K)
    row_w = jnp.zeros((CAP,), jnp.float32).at[pos].set(w_flat)

    xs = x2d[row_token]

    ysw = _gmm(tile_expert, xs, W1.astype(jnp.bfloat16), b1.reshape(_E, 1, U1),
               W2.astype(jnp.bfloat16), b2.reshape(_E, 1, U2),
               row_w.reshape(NT, 1, _TM))

    out = ysw[pos.reshape(T, _K)].sum(axis=1)
    return out.reshape(B, S, U2)


# speedup vs baseline: 2.4635x; 1.1005x over previous
import functools

import jax
import jax.numpy as jnp
from jax.experimental import pallas as pl
from jax.experimental.pallas import tpu as pltpu

_INTERPRET = False

_E = 8
_K = 2
_TM = 256


def _erf(x):
    return jax.lax.erf(x)


def _gelu(x):
    return 0.5 * x * (1.0 + _erf(x * 0.7071067811865476))


def _router_body(x_ref, wg_ref, bg_ref, idx_ref, w_ref):
    x = x_ref[...]
    wg = wg_ref[...]
    logits = jnp.dot(x, wg, preferred_element_type=jnp.float32) + bg_ref[...][None, :]
    m = jnp.max(logits, axis=-1, keepdims=True)
    p = jnp.exp(logits - m)
    g = p / jnp.sum(p, axis=-1, keepdims=True)
    lane = jax.lax.broadcasted_iota(jnp.int32, g.shape, 1)
    v0 = jnp.max(g, axis=-1, keepdims=True)
    e0 = jnp.min(jnp.where(g == v0, lane, _E), axis=-1, keepdims=True)
    g2 = jnp.where(lane == e0, -1.0, g)
    v1 = jnp.max(g2, axis=-1, keepdims=True)
    e1 = jnp.min(jnp.where(g2 == v1, lane, _E), axis=-1, keepdims=True)
    s = v0 + v1 + 1e-9
    idx_ref[...] = jnp.concatenate([e0, e1], axis=1)
    w_ref[...] = jnp.concatenate([v0 / s, v1 / s], axis=1)


def _router(x2d, Wg, bg):
    T, D = x2d.shape
    TMR = 512
    return pl.pallas_call(
        _router_body,
        grid=(T // TMR,),
        in_specs=[
            pl.BlockSpec((TMR, D), lambda i: (i, 0)),
            pl.BlockSpec((D, _E), lambda i: (0, 0)),
            pl.BlockSpec((_E,), lambda i: (0,)),
        ],
        out_specs=[
            pl.BlockSpec((TMR, _K), lambda i: (i, 0)),
            pl.BlockSpec((TMR, _K), lambda i: (i, 0)),
        ],
        out_shape=[
            jax.ShapeDtypeStruct((T, _K), jnp.int32),
            jax.ShapeDtypeStruct((T, _K), jnp.float32),
        ],
        interpret=_INTERPRET,
    )(x2d, Wg, bg)


def _gmm_body(te_ref, xs_ref, w1_ref, b1_ref, w2_ref, b2_ref, rw_ref, out_ref,
              w1b_ref, w2b_ref):
    i = pl.program_id(0)
    new_w = jnp.logical_or(i == 0, te_ref[i] != te_ref[jnp.maximum(i - 1, 0)])

    @pl.when(new_w)
    def _():
        w1b_ref[...] = w1_ref[0].astype(jnp.bfloat16)
        w2b_ref[...] = w2_ref[0].astype(jnp.bfloat16)

    x = xs_ref[...].astype(jnp.bfloat16)
    h = jnp.dot(x, w1b_ref[...], preferred_element_type=jnp.float32) + b1_ref[0, 0][None, :]
    h = _gelu(h)
    y = jnp.dot(h.astype(jnp.bfloat16), w2b_ref[...], preferred_element_type=jnp.float32)
    y = _gelu(y + b2_ref[0, 0][None, :])
    out_ref[...] = y * rw_ref[0, 0][:, None]


def _gmm(tile_expert, xs, W1b, b1, W2b, b2, row_w3):
    CAP, D = xs.shape
    U1 = W1b.shape[2]
    U2 = W2b.shape[2]
    NT = CAP // _TM
    grid_spec = pltpu.PrefetchScalarGridSpec(
        num_scalar_prefetch=1,
        grid=(NT,),
        in_specs=[
            pl.BlockSpec((_TM, D), lambda i, te: (i, 0)),
            pl.BlockSpec((1, D, U1), lambda i, te: (te[i], 0, 0)),
            pl.BlockSpec((1, 1, U1), lambda i, te: (te[i], 0, 0)),
            pl.BlockSpec((1, U1, U2), lambda i, te: (te[i], 0, 0)),
            pl.BlockSpec((1, 1, U2), lambda i, te: (te[i], 0, 0)),
            pl.BlockSpec((1, 1, _TM), lambda i, te: (i, 0, 0)),
        ],
        out_specs=pl.BlockSpec((_TM, U2), lambda i, te: (i, 0)),
        scratch_shapes=[
            pltpu.VMEM((D, U1), jnp.bfloat16),
            pltpu.VMEM((U1, U2), jnp.bfloat16),
        ],
    )
    return pl.pallas_call(
        _gmm_body,
        grid_spec=grid_spec,
        out_shape=jax.ShapeDtypeStruct((CAP, U2), jnp.float32),
        interpret=_INTERPRET,
    )(tile_expert, xs, W1b, b1, W2b, b2, row_w3)


def kernel(inputs, Wg, bg, W1, b1, W2, b2):
    B, S, D = inputs.shape
    T = B * S
    U1 = W1.shape[2]
    U2 = W2.shape[2]
    x2d = inputs.reshape(T, D)

    idx_pair, w_pair = _router(x2d, Wg, bg)

    lane = jnp.arange(_E, dtype=jnp.int32)
    expert_flat = idx_pair.reshape(-1)
    w_flat = w_pair.reshape(-1)

    match = (expert_flat[:, None] == lane[None, :]).astype(jnp.int32)
    ranks_all = jnp.cumsum(match, axis=0) - 1
    rank = jnp.take_along_axis(ranks_all, expert_flat[:, None], axis=1)[:, 0]
    counts = jnp.sum(match, axis=0)
    padded = ((counts + _TM - 1) // _TM) * _TM
    offs = jnp.concatenate([jnp.zeros((1,), jnp.int32),
                            jnp.cumsum(padded)[:-1].astype(jnp.int32)])
    pos = offs[expert_flat] + rank

    CAP = _K * T + _E * _TM
    NT = CAP // _TM
    ends = (offs + padded) // _TM
    tiles = jnp.arange(NT, dtype=jnp.int32)
    tile_expert = jnp.minimum(
        jnp.sum(tiles[:, None] >= ends[None, :], axis=1), _E - 1
    ).astype(jnp.int32)

    row_token = jnp.zeros((CAP,), jnp.int32).at[pos].set(
        jnp.arange(_K * T, dtype=jnp.int32) // _K)
    row_w = jnp.zeros((CAP,), jnp.float32).at[pos].set(w_flat)

    xs = x2d[row_token]

    ysw = _gmm(tile_expert, xs, W1, b1.reshape(_E, 1, U1),
               W2, b2.reshape(_E, 1, U2),
               row_w.reshape(NT, 1, _TM))

    out = ysw[pos.reshape(T, _K)].sum(axis=1)
    return out.reshape(B, S, U2)


# speedup vs baseline: 2.4898x; 1.0107x over previous
import functools

import jax
import jax.numpy as jnp
from jax.experimental import pallas as pl
from jax.experimental.pallas import tpu as pltpu

_INTERPRET = False

_E = 8
_K = 2
_TM = 256


def _erf(x):
    return jax.lax.erf(x)


def _gelu(x):
    return 0.5 * x * (1.0 + _erf(x * 0.7071067811865476))


def _router_body(x_ref, wg_ref, bg_ref, idx_ref, w_ref):
    x = x_ref[...]
    wg = wg_ref[...]
    logits = jnp.dot(x, wg, preferred_element_type=jnp.float32) + bg_ref[...][None, :]
    m = jnp.max(logits, axis=-1, keepdims=True)
    p = jnp.exp(logits - m)
    g = p / jnp.sum(p, axis=-1, keepdims=True)
    lane = jax.lax.broadcasted_iota(jnp.int32, g.shape, 1)
    v0 = jnp.max(g, axis=-1, keepdims=True)
    e0 = jnp.min(jnp.where(g == v0, lane, _E), axis=-1, keepdims=True)
    g2 = jnp.where(lane == e0, -1.0, g)
    v1 = jnp.max(g2, axis=-1, keepdims=True)
    e1 = jnp.min(jnp.where(g2 == v1, lane, _E), axis=-1, keepdims=True)
    s = v0 + v1 + 1e-9
    idx_ref[...] = jnp.concatenate([e0, e1], axis=1)
    w_ref[...] = jnp.concatenate([v0 / s, v1 / s], axis=1)


def _router(x2d, Wg, bg):
    T, D = x2d.shape
    TMR = 512
    return pl.pallas_call(
        _router_body,
        grid=(T // TMR,),
        in_specs=[
            pl.BlockSpec((TMR, D), lambda i: (i, 0)),
            pl.BlockSpec((D, _E), lambda i: (0, 0)),
            pl.BlockSpec((_E,), lambda i: (0,)),
        ],
        out_specs=[
            pl.BlockSpec((TMR, _K), lambda i: (i, 0)),
            pl.BlockSpec((TMR, _K), lambda i: (i, 0)),
        ],
        out_shape=[
            jax.ShapeDtypeStruct((T, _K), jnp.int32),
            jax.ShapeDtypeStruct((T, _K), jnp.float32),
        ],
        interpret=_INTERPRET,
    )(x2d, Wg, bg)


def _gmm_body(te_ref, xs_ref, w1_ref, b1_ref, w2_ref, b2_ref, rw_ref, out_ref,
              w1b_ref, w2b_ref):
    i = pl.program_id(0)
    new_w = jnp.logical_or(i == 0, te_ref[i] != te_ref[jnp.maximum(i - 1, 0)])

    @pl.when(new_w)
    def _():
        w1b_ref[...] = w1_ref[0].astype(jnp.bfloat16)
        w2b_ref[...] = w2_ref[0].astype(jnp.bfloat16)

    x = xs_ref[...].astype(jnp.bfloat16)
    h = jnp.dot(x, w1b_ref[...], preferred_element_type=jnp.float32) + b1_ref[0, 0][None, :]
    h = _gelu(h)
    y = jnp.dot(h.astype(jnp.bfloat16), w2b_ref[...], preferred_element_type=jnp.float32)
    y = _gelu(y + b2_ref[0, 0][None, :])
    out_ref[...] = y * rw_ref[0, 0][:, None]


def _gmm(tile_expert, xs, W1b, b1, W2b, b2, row_w3):
    CAP, D = xs.shape
    U1 = W1b.shape[2]
    U2 = W2b.shape[2]
    NT = CAP // _TM
    grid_spec = pltpu.PrefetchScalarGridSpec(
        num_scalar_prefetch=1,
        grid=(NT,),
        in_specs=[
            pl.BlockSpec((_TM, D), lambda i, te: (i, 0)),
            pl.BlockSpec((1, D, U1), lambda i, te: (te[i], 0, 0)),
            pl.BlockSpec((1, 1, U1), lambda i, te: (te[i], 0, 0)),
            pl.BlockSpec((1, U1, U2), lambda i, te: (te[i], 0, 0)),
            pl.BlockSpec((1, 1, U2), lambda i, te: (te[i], 0, 0)),
            pl.BlockSpec((1, 1, _TM), lambda i, te: (i, 0, 0)),
        ],
        out_specs=pl.BlockSpec((_TM, U2), lambda i, te: (i, 0)),
        scratch_shapes=[
            pltpu.VMEM((D, U1), jnp.bfloat16),
            pltpu.VMEM((U1, U2), jnp.bfloat16),
        ],
    )
    return pl.pallas_call(
        _gmm_body,
        grid_spec=grid_spec,
        out_shape=jax.ShapeDtypeStruct((CAP, U2), jnp.float32),
        interpret=_INTERPRET,
    )(tile_expert, xs, W1b, b1, W2b, b2, row_w3)


def kernel(inputs, Wg, bg, W1, b1, W2, b2):
    B, S, D = inputs.shape
    T = B * S
    U1 = W1.shape[2]
    U2 = W2.shape[2]
    x2d = inputs.reshape(T, D)

    idx_pair, w_pair = _router(x2d, Wg, bg)

    lane = jnp.arange(_E, dtype=jnp.int32)
    expert_flat = idx_pair.reshape(-1)
    w_flat = w_pair.reshape(-1)

    match_t = (lane[:, None] == expert_flat[None, :]).astype(jnp.int32)
    ranks_t = jnp.cumsum(match_t, axis=1)
    rank = jnp.sum(match_t * ranks_t, axis=0) - 1
    counts = ranks_t[:, -1]
    padded = ((counts + _TM - 1) // _TM) * _TM
    offs = jnp.concatenate([jnp.zeros((1,), jnp.int32),
                            jnp.cumsum(padded)[:-1].astype(jnp.int32)])
    pos = offs[expert_flat] + rank

    CAP = _K * T + _E * _TM
    NT = CAP // _TM
    ends = (offs + padded) // _TM
    tiles = jnp.arange(NT, dtype=jnp.int32)
    tile_expert = jnp.minimum(
        jnp.sum(tiles[:, None] >= ends[None, :], axis=1), _E - 1
    ).astype(jnp.int32)

    row_token = jnp.zeros((CAP,), jnp.int32).at[pos].set(
        jnp.arange(_K * T, dtype=jnp.int32) // _K)
    row_w = jnp.zeros((CAP,), jnp.float32).at[pos].set(w_flat)

    xs = x2d[row_token]

    ysw = _gmm(tile_expert, xs, W1, b1.reshape(_E, 1, U1),
               W2, b2.reshape(_E, 1, U2),
               row_w.reshape(NT, 1, _TM))

    out = ysw[pos.reshape(T, _K)].sum(axis=1)
    return out.reshape(B, S, U2)


# speedup vs baseline: 2.8417x; 1.1413x over previous
import functools

import jax
import jax.numpy as jnp
from jax import lax
from jax.experimental import pallas as pl
from jax.experimental.pallas import tpu as pltpu
from jax.experimental.pallas import tpu_sc as plsc

_INTERPRET = False

_E = 8
_K = 2
_TM = 256


def _erf(x):
    return jax.lax.erf(x)


def _gelu(x):
    return 0.5 * x * (1.0 + _erf(x * 0.7071067811865476))


def _router_body(x_ref, wg_ref, bg_ref, idx_ref, w_ref):
    x = x_ref[...]
    wg = wg_ref[...]
    logits = jnp.dot(x, wg, preferred_element_type=jnp.float32) + bg_ref[...][None, :]
    m = jnp.max(logits, axis=-1, keepdims=True)
    p = jnp.exp(logits - m)
    g = p / jnp.sum(p, axis=-1, keepdims=True)
    lane = jax.lax.broadcasted_iota(jnp.int32, g.shape, 1)
    v0 = jnp.max(g, axis=-1, keepdims=True)
    e0 = jnp.min(jnp.where(g == v0, lane, _E), axis=-1, keepdims=True)
    g2 = jnp.where(lane == e0, -1.0, g)
    v1 = jnp.max(g2, axis=-1, keepdims=True)
    e1 = jnp.min(jnp.where(g2 == v1, lane, _E), axis=-1, keepdims=True)
    s = v0 + v1 + 1e-9
    idx_ref[...] = jnp.concatenate([e0, e1], axis=1)
    w_ref[...] = jnp.concatenate([v0 / s, v1 / s], axis=1)


def _router(x2d, Wg, bg):
    T, D = x2d.shape
    TMR = 512
    return pl.pallas_call(
        _router_body,
        grid=(T // TMR,),
        in_specs=[
            pl.BlockSpec((TMR, D), lambda i: (i, 0)),
            pl.BlockSpec((D, _E), lambda i: (0, 0)),
            pl.BlockSpec((_E,), lambda i: (0,)),
        ],
        out_specs=[
            pl.BlockSpec((TMR, _K), lambda i: (i, 0)),
            pl.BlockSpec((TMR, _K), lambda i: (i, 0)),
        ],
        out_shape=[
            jax.ShapeDtypeStruct((T, _K), jnp.int32),
            jax.ShapeDtypeStruct((T, _K), jnp.float32),
        ],
        interpret=_INTERPRET,
    )(x2d, Wg, bg)


def _gmm_body(te_ref, xs_ref, w1_ref, b1_ref, w2_ref, b2_ref, rw_ref, out_ref,
              w1b_ref, w2b_ref):
    i = pl.program_id(0)
    new_w = jnp.logical_or(i == 0, te_ref[i] != te_ref[jnp.maximum(i - 1, 0)])

    @pl.when(new_w)
    def _():
        w1b_ref[...] = w1_ref[0].astype(jnp.bfloat16)
        w2b_ref[...] = w2_ref[0].astype(jnp.bfloat16)

    x = xs_ref[...].astype(jnp.bfloat16)
    h = jnp.dot(x, w1b_ref[...], preferred_element_type=jnp.float32) + b1_ref[0, 0][None, :]
    h = _gelu(h)
    y = jnp.dot(h.astype(jnp.bfloat16), w2b_ref[...], preferred_element_type=jnp.float32)
    y = _gelu(y + b2_ref[0, 0][None, :])
    out_ref[...] = y * rw_ref[0, 0][:, None]


def _gmm(tile_expert, xs, W1b, b1, W2b, b2, row_w3):
    CAP, D = xs.shape
    U1 = W1b.shape[2]
    U2 = W2b.shape[2]
    NT = CAP // _TM
    grid_spec = pltpu.PrefetchScalarGridSpec(
        num_scalar_prefetch=1,
        grid=(NT,),
        in_specs=[
            pl.BlockSpec((_TM, D), lambda i, te: (i, 0)),
            pl.BlockSpec((1, D, U1), lambda i, te: (te[i], 0, 0)),
            pl.BlockSpec((1, 1, U1), lambda i, te: (te[i], 0, 0)),
            pl.BlockSpec((1, U1, U2), lambda i, te: (te[i], 0, 0)),
            pl.BlockSpec((1, 1, U2), lambda i, te: (te[i], 0, 0)),
            pl.BlockSpec((1, 1, _TM), lambda i, te: (i, 0, 0)),
        ],
        out_specs=pl.BlockSpec((_TM, U2), lambda i, te: (i, 0)),
        scratch_shapes=[
            pltpu.VMEM((D, U1), jnp.bfloat16),
            pltpu.VMEM((U1, U2), jnp.bfloat16),
        ],
    )
    return pl.pallas_call(
        _gmm_body,
        grid_spec=grid_spec,
        out_shape=jax.ShapeDtypeStruct((CAP, U2), jnp.float32),
        interpret=_INTERPRET,
    )(tile_expert, xs, W1b, b1, W2b, b2, row_w3)


def _combine(ysw, pos_flat, T, U2):
    NW = 32
    per = T // NW
    CT = 16
    nch = per // CT
    mesh = plsc.VectorSubcoreMesh(core_axis_name="c", subcore_axis_name="s")

    @functools.partial(
        pl.kernel,
        mesh=mesh,
        out_type=jax.ShapeDtypeStruct((T, U2), jnp.float32),
        scratch_types=[
            pltpu.VMEM((2 * CT,), jnp.int32),
            pltpu.VMEM((2 * CT, U2), jnp.float32),
            pltpu.VMEM((CT, U2), jnp.float32),
            pltpu.SemaphoreType.DMA,
        ],
    )
    def k(ysw_hbm, pos_hbm, out_hbm, idx_v, rows_v, out_v, sem):
        wid = lax.axis_index("s") * 2 + lax.axis_index("c")
        tok0 = wid * per

        def chunk(c, _):
            pltpu.sync_copy(pos_hbm.at[pl.ds((tok0 + c * CT) * 2, 2 * CT)], idx_v)
            pltpu.async_copy(ysw_hbm.at[idx_v], rows_v, sem).wait()

            def tok(j, _):
                def col(cc, _):
                    sl = pl.ds(cc * 16, 16)
                    out_v[j, sl] = rows_v[2 * j, sl] + rows_v[2 * j + 1, sl]
                    return 0
                lax.fori_loop(0, U2 // 16, col, 0)
                return 0
            lax.fori_loop(0, CT, tok, 0)
            pltpu.sync_copy(out_v, out_hbm.at[pl.ds(tok0 + c * CT, CT)])
            return 0
        lax.fori_loop(0, nch, chunk, 0)

    return k(ysw, pos_flat)


def kernel(inputs, Wg, bg, W1, b1, W2, b2):
    B, S, D = inputs.shape
    T = B * S
    U1 = W1.shape[2]
    U2 = W2.shape[2]
    x2d = inputs.reshape(T, D)

    idx_pair, w_pair = _router(x2d, Wg, bg)

    lane = jnp.arange(_E, dtype=jnp.int32)
    expert_flat = idx_pair.reshape(-1)
    w_flat = w_pair.reshape(-1)

    match_t = (lane[:, None] == expert_flat[None, :]).astype(jnp.int32)
    ranks_t = jnp.cumsum(match_t, axis=1)
    rank = jnp.sum(match_t * ranks_t, axis=0) - 1
    counts = ranks_t[:, -1]
    padded = ((counts + _TM - 1) // _TM) * _TM
    offs = jnp.concatenate([jnp.zeros((1,), jnp.int32),
                            jnp.cumsum(padded)[:-1].astype(jnp.int32)])
    pos = offs[expert_flat] + rank

    CAP = _K * T + _E * _TM
    NT = CAP // _TM
    ends = (offs + padded) // _TM
    tiles = jnp.arange(NT, dtype=jnp.int32)
    tile_expert = jnp.minimum(
        jnp.sum(tiles[:, None] >= ends[None, :], axis=1), _E - 1
    ).astype(jnp.int32)

    row_token = jnp.zeros((CAP,), jnp.int32).at[pos].set(
        jnp.arange(_K * T, dtype=jnp.int32) // _K)
    row_w = jnp.zeros((CAP,), jnp.float32).at[pos].set(w_flat)

    xs = x2d[row_token]

    ysw = _gmm(tile_expert, xs, W1, b1.reshape(_E, 1, U1),
               W2, b2.reshape(_E, 1, U2),
               row_w.reshape(NT, 1, _TM))

    out = _combine(ysw, pos, T, U2)
    return out.reshape(B, S, U2)


# speedup vs baseline: 2.8608x; 1.0067x over previous
import functools

import jax
import jax.numpy as jnp
from jax import lax
from jax.experimental import pallas as pl
from jax.experimental.pallas import tpu as pltpu
from jax.experimental.pallas import tpu_sc as plsc

_INTERPRET = False

_E = 8
_K = 2
_TM = 256


def _erf(x):
    return jax.lax.erf(x)


def _gelu(x):
    return 0.5 * x * (1.0 + _erf(x * 0.7071067811865476))


def _router_body(x_ref, wg_ref, bg_ref, idx_ref, w_ref):
    x = x_ref[...]
    wg = wg_ref[...]
    logits = jnp.dot(x, wg, preferred_element_type=jnp.float32) + bg_ref[...][None, :]
    m = jnp.max(logits, axis=-1, keepdims=True)
    p = jnp.exp(logits - m)
    g = p / jnp.sum(p, axis=-1, keepdims=True)
    lane = jax.lax.broadcasted_iota(jnp.int32, g.shape, 1)
    v0 = jnp.max(g, axis=-1, keepdims=True)
    e0 = jnp.min(jnp.where(g == v0, lane, _E), axis=-1, keepdims=True)
    g2 = jnp.where(lane == e0, -1.0, g)
    v1 = jnp.max(g2, axis=-1, keepdims=True)
    e1 = jnp.min(jnp.where(g2 == v1, lane, _E), axis=-1, keepdims=True)
    s = v0 + v1 + 1e-9
    idx_ref[...] = jnp.concatenate([e0, e1], axis=1)
    w_ref[...] = jnp.concatenate([v0 / s, v1 / s], axis=1)


def _router(x2d, Wg, bg):
    T, D = x2d.shape
    TMR = 512
    return pl.pallas_call(
        _router_body,
        grid=(T // TMR,),
        in_specs=[
            pl.BlockSpec((TMR, D), lambda i: (i, 0)),
            pl.BlockSpec((D, _E), lambda i: (0, 0)),
            pl.BlockSpec((_E,), lambda i: (0,)),
        ],
        out_specs=[
            pl.BlockSpec((TMR, _K), lambda i: (i, 0)),
            pl.BlockSpec((TMR, _K), lambda i: (i, 0)),
        ],
        out_shape=[
            jax.ShapeDtypeStruct((T, _K), jnp.int32),
            jax.ShapeDtypeStruct((T, _K), jnp.float32),
        ],
        interpret=_INTERPRET,
    )(x2d, Wg, bg)


def _gmm_body(te_ref, xs_ref, w1_ref, b1_ref, w2_ref, b2_ref, rw_ref, out_ref,
              w1b_ref, w2b_ref):
    i = pl.program_id(0)
    new_w = jnp.logical_or(i == 0, te_ref[i] != te_ref[jnp.maximum(i - 1, 0)])

    @pl.when(new_w)
    def _():
        w1b_ref[...] = w1_ref[0].astype(jnp.bfloat16)
        w2b_ref[...] = w2_ref[0].astype(jnp.bfloat16)

    x = xs_ref[...]
    h = jnp.dot(x, w1b_ref[...], preferred_element_type=jnp.float32) + b1_ref[0, 0][None, :]
    h = _gelu(h)
    y = jnp.dot(h.astype(jnp.bfloat16), w2b_ref[...], preferred_element_type=jnp.float32)
    y = _gelu(y + b2_ref[0, 0][None, :])
    out_ref[...] = y * rw_ref[0, 0][:, None]


def _gmm(tile_expert, xs, W1b, b1, W2b, b2, row_w3):
    CAP, D = xs.shape
    U1 = W1b.shape[2]
    U2 = W2b.shape[2]
    NT = CAP // _TM
    grid_spec = pltpu.PrefetchScalarGridSpec(
        num_scalar_prefetch=1,
        grid=(NT,),
        in_specs=[
            pl.BlockSpec((_TM, D), lambda i, te: (i, 0)),
            pl.BlockSpec((1, D, U1), lambda i, te: (te[i], 0, 0)),
            pl.BlockSpec((1, 1, U1), lambda i, te: (te[i], 0, 0)),
            pl.BlockSpec((1, U1, U2), lambda i, te: (te[i], 0, 0)),
            pl.BlockSpec((1, 1, U2), lambda i, te: (te[i], 0, 0)),
            pl.BlockSpec((1, 1, _TM), lambda i, te: (i, 0, 0)),
        ],
        out_specs=pl.BlockSpec((_TM, U2), lambda i, te: (i, 0)),
        scratch_shapes=[
            pltpu.VMEM((D, U1), jnp.bfloat16),
            pltpu.VMEM((U1, U2), jnp.bfloat16),
        ],
    )
    return pl.pallas_call(
        _gmm_body,
        grid_spec=grid_spec,
        out_shape=jax.ShapeDtypeStruct((CAP, U2), jnp.float32),
        interpret=_INTERPRET,
    )(tile_expert, xs, W1b, b1, W2b, b2, row_w3)


def _combine(ysw, pos_flat, T, U2):
    NW = 32
    per = T // NW
    CT = 32
    nch = per // CT
    mesh = plsc.VectorSubcoreMesh(core_axis_name="c", subcore_axis_name="s")

    @functools.partial(
        pl.kernel,
        mesh=mesh,
        out_type=jax.ShapeDtypeStruct((T, U2), jnp.float32),
        scratch_types=[
            pltpu.VMEM((2 * CT,), jnp.int32),
            pltpu.VMEM((2 * CT, U2), jnp.float32),
            pltpu.VMEM((CT, U2), jnp.float32),
            pltpu.SemaphoreType.DMA,
        ],
    )
    def k(ysw_hbm, pos_hbm, out_hbm, idx_v, rows_v, out_v, sem):
        wid = lax.axis_index("s") * 2 + lax.axis_index("c")
        tok0 = wid * per

        def chunk(c, _):
            pltpu.sync_copy(pos_hbm.at[pl.ds((tok0 + c * CT) * 2, 2 * CT)], idx_v)
            pltpu.async_copy(ysw_hbm.at[idx_v], rows_v, sem).wait()

            def tok(j, _):
                def col(cc, _):
                    for u in range(4):
                        sl = pl.ds(cc * 64 + u * 16, 16)
                        out_v[j, sl] = rows_v[2 * j, sl] + rows_v[2 * j + 1, sl]
                    return 0
                lax.fori_loop(0, U2 // 64, col, 0)
                return 0
            lax.fori_loop(0, CT, tok, 0)
            pltpu.sync_copy(out_v, out_hbm.at[pl.ds(tok0 + c * CT, CT)])
            return 0
        lax.fori_loop(0, nch, chunk, 0)

    return k(ysw, pos_flat)


def kernel(inputs, Wg, bg, W1, b1, W2, b2):
    B, S, D = inputs.shape
    T = B * S
    U1 = W1.shape[2]
    U2 = W2.shape[2]
    x2d = inputs.reshape(T, D)

    idx_pair, w_pair = _router(x2d, Wg, bg)

    lane = jnp.arange(_E, dtype=jnp.int32)
    expert_flat = idx_pair.reshape(-1)

    match_t = (lane[:, None] == expert_flat[None, :]).astype(jnp.int32)
    ranks_t = jnp.cumsum(match_t, axis=1)
    rank = jnp.sum(match_t * ranks_t, axis=0) - 1
    counts = ranks_t[:, -1]
    padded = ((counts + _TM - 1) // _TM) * _TM
    offs = jnp.concatenate([jnp.zeros((1,), jnp.int32),
                            jnp.cumsum(padded)[:-1].astype(jnp.int32)])
    pos = offs[expert_flat] + rank

    CAP = _K * T + _E * _TM
    NT = CAP // _TM
    ends = (offs + padded) // _TM
    tiles = jnp.arange(NT, dtype=jnp.int32)
    tile_expert = jnp.minimum(
        jnp.sum(tiles[:, None] >= ends[None, :], axis=1), _E - 1
    ).astype(jnp.int32)

    row_token = jnp.zeros((CAP,), jnp.int32).at[pos].set(
        jnp.arange(_K * T, dtype=jnp.int32) // _K)
    row_w = jnp.zeros((CAP,), jnp.float32).at[pos].set(w_pair.reshape(-1))

    xs = x2d.astype(jnp.bfloat16)[row_token]

    ysw = _gmm(tile_expert, xs, W1, b1.reshape(_E, 1, U1),
               W2, b2.reshape(_E, 1, U2),
               row_w.reshape(NT, 1, _TM))

    out = _combine(ysw, pos, T, U2)
    return out.reshape(B, S, U2)


# speedup vs baseline: 2.9301x; 1.0242x over previous
import functools

import jax
import jax.numpy as jnp
from jax import lax
from jax.experimental import pallas as pl
from jax.experimental.pallas import tpu as pltpu
from jax.experimental.pallas import tpu_sc as plsc

_INTERPRET = False

_E = 8
_K = 2
_TM = 256


def _erf(x):
    return jax.lax.erf(x)


def _gelu(x):
    return 0.5 * x * (1.0 + _erf(x * 0.7071067811865476))


def _router_body(x_ref, wg_ref, bg_ref, idx_ref, w_ref):
    x = x_ref[...]
    wg = wg_ref[...]
    logits = jnp.dot(x, wg, preferred_element_type=jnp.float32) + bg_ref[...][None, :]
    m = jnp.max(logits, axis=-1, keepdims=True)
    p = jnp.exp(logits - m)
    g = p / jnp.sum(p, axis=-1, keepdims=True)
    lane = jax.lax.broadcasted_iota(jnp.int32, g.shape, 1)
    v0 = jnp.max(g, axis=-1, keepdims=True)
    e0 = jnp.min(jnp.where(g == v0, lane, _E), axis=-1, keepdims=True)
    g2 = jnp.where(lane == e0, -1.0, g)
    v1 = jnp.max(g2, axis=-1, keepdims=True)
    e1 = jnp.min(jnp.where(g2 == v1, lane, _E), axis=-1, keepdims=True)
    s = v0 + v1 + 1e-9
    idx_ref[...] = jnp.concatenate([e0, e1], axis=1)
    w_ref[...] = jnp.concatenate([v0 / s, v1 / s], axis=1)


def _router(x2d, Wg, bg):
    T, D = x2d.shape
    TMR = 512
    return pl.pallas_call(
        _router_body,
        grid=(T // TMR,),
        in_specs=[
            pl.BlockSpec((TMR, D), lambda i: (i, 0)),
            pl.BlockSpec((D, _E), lambda i: (0, 0)),
            pl.BlockSpec((_E,), lambda i: (0,)),
        ],
        out_specs=[
            pl.BlockSpec((TMR, _K), lambda i: (i, 0)),
            pl.BlockSpec((TMR, _K), lambda i: (i, 0)),
        ],
        out_shape=[
            jax.ShapeDtypeStruct((T, _K), jnp.int32),
            jax.ShapeDtypeStruct((T, _K), jnp.float32),
        ],
        interpret=_INTERPRET,
    )(x2d, Wg, bg)


def _gmm_body(te_ref, xs_ref, w1_ref, b1_ref, w2_ref, b2_ref, rw_ref, out_ref,
              w1b_ref, w2b_ref):
    i = pl.program_id(0)
    new_w = jnp.logical_or(i == 0, te_ref[i] != te_ref[jnp.maximum(i - 1, 0)])

    @pl.when(new_w)
    def _():
        w1b_ref[...] = w1_ref[0].astype(jnp.bfloat16)
        w2b_ref[...] = w2_ref[0].astype(jnp.bfloat16)

    x = xs_ref[...]
    h = jnp.dot(x, w1b_ref[...], preferred_element_type=jnp.float32) + b1_ref[0, 0][None, :]
    h = _gelu(h)
    y = jnp.dot(h.astype(jnp.bfloat16), w2b_ref[...], preferred_element_type=jnp.float32)
    y = _gelu(y + b2_ref[0, 0][None, :])
    out_ref[...] = y * rw_ref[0, 0][:, None]


def _gmm(tile_expert, xs, W1b, b1, W2b, b2, row_w3):
    CAP, D = xs.shape
    U1 = W1b.shape[2]
    U2 = W2b.shape[2]
    NT = CAP // _TM
    grid_spec = pltpu.PrefetchScalarGridSpec(
        num_scalar_prefetch=1,
        grid=(NT,),
        in_specs=[
            pl.BlockSpec((_TM, D), lambda i, te: (i, 0)),
            pl.BlockSpec((1, D, U1), lambda i, te: (te[i], 0, 0)),
            pl.BlockSpec((1, 1, U1), lambda i, te: (te[i], 0, 0)),
            pl.BlockSpec((1, U1, U2), lambda i, te: (te[i], 0, 0)),
            pl.BlockSpec((1, 1, U2), lambda i, te: (te[i], 0, 0)),
            pl.BlockSpec((1, 1, _TM), lambda i, te: (i, 0, 0)),
        ],
        out_specs=pl.BlockSpec((_TM, U2), lambda i, te: (i, 0)),
        scratch_shapes=[
            pltpu.VMEM((D, U1), jnp.bfloat16),
            pltpu.VMEM((U1, U2), jnp.bfloat16),
        ],
    )
    return pl.pallas_call(
        _gmm_body,
        grid_spec=grid_spec,
        out_shape=jax.ShapeDtypeStruct((CAP, U2), jnp.float32),
        interpret=_INTERPRET,
    )(tile_expert, xs, W1b, b1, W2b, b2, row_w3)


def _combine(ysw, pos_flat, T, U2):
    NW = 32
    per = T // NW
    CT = 16
    nch = per // CT
    mesh = plsc.VectorSubcoreMesh(core_axis_name="c", subcore_axis_name="s")

    @functools.partial(
        pl.kernel,
        mesh=mesh,
        out_type=jax.ShapeDtypeStruct((T, U2), jnp.float32),
        scratch_types=[
            pltpu.VMEM((nch, 2 * CT), jnp.int32),
            pltpu.VMEM((2 * CT, U2), jnp.float32),
            pltpu.VMEM((2 * CT, U2), jnp.float32),
            pltpu.VMEM((CT, U2), jnp.float32),
            pltpu.SemaphoreType.DMA,
            pltpu.SemaphoreType.DMA,
        ],
    )
    def k(ysw_hbm, pos_hbm, out_hbm, idx_v, rows_a, rows_b, out_v, sem_a, sem_b):
        wid = lax.axis_index("s") * 2 + lax.axis_index("c")
        tok0 = wid * per
        pltpu.sync_copy(pos_hbm.at[wid], idx_v)
        bufs = [(rows_a, sem_a), (rows_b, sem_b)]
        handles = [None] * nch
        handles[0] = pltpu.async_copy(ysw_hbm.at[idx_v.at[0]], rows_a, sem_a)
        for c in range(nch):
            rows_v, _ = bufs[c % 2]
            nrows, nsem = bufs[(c + 1) % 2]
            if c + 1 < nch:
                handles[c + 1] = pltpu.async_copy(
                    ysw_hbm.at[idx_v.at[c + 1]], nrows, nsem)
            handles[c].wait()

            def tok(j, _):
                def col(cc, _):
                    for u in range(4):
                        sl = pl.ds(cc * 64 + u * 16, 16)
                        out_v[j, sl] = rows_v[2 * j, sl] + rows_v[2 * j + 1, sl]
                    return 0
                lax.fori_loop(0, U2 // 64, col, 0)
                return 0
            lax.fori_loop(0, CT, tok, 0)
            pltpu.sync_copy(out_v, out_hbm.at[pl.ds(tok0 + c * CT, CT)])

    return k(ysw, pos_flat.reshape(NW, nch, 2 * CT))


def kernel(inputs, Wg, bg, W1, b1, W2, b2):
    B, S, D = inputs.shape
    T = B * S
    U1 = W1.shape[2]
    U2 = W2.shape[2]
    x2d = inputs.reshape(T, D)

    idx_pair, w_pair = _router(x2d, Wg, bg)

    lane = jnp.arange(_E, dtype=jnp.int32)
    expert_flat = idx_pair.reshape(-1)

    match_t = (lane[:, None] == expert_flat[None, :]).astype(jnp.int32)
    ranks_t = jnp.cumsum(match_t, axis=1)
    rank = jnp.sum(match_t * ranks_t, axis=0) - 1
    counts = ranks_t[:, -1]
    padded = ((counts + _TM - 1) // _TM) * _TM
    offs = jnp.concatenate([jnp.zeros((1,), jnp.int32),
                            jnp.cumsum(padded)[:-1].astype(jnp.int32)])
    pos = offs[expert_flat] + rank

    CAP = _K * T + _E * _TM
    NT = CAP // _TM
    ends = (offs + padded) // _TM
    tiles = jnp.arange(NT, dtype=jnp.int32)
    tile_expert = jnp.minimum(
        jnp.sum(tiles[:, None] >= ends[None, :], axis=1), _E - 1
    ).astype(jnp.int32)

    row_token = jnp.zeros((CAP,), jnp.int32).at[pos].set(
        jnp.arange(_K * T, dtype=jnp.int32) // _K)
    row_w = jnp.zeros((CAP,), jnp.float32).at[pos].set(w_pair.reshape(-1))

    xs = x2d.astype(jnp.bfloat16)[row_token]

    ysw = _gmm(tile_expert, xs, W1, b1.reshape(_E, 1, U1),
               W2, b2.reshape(_E, 1, U2),
               row_w.reshape(NT, 1, _TM))

    out = _combine(ysw, pos, T, U2)
    return out.reshape(B, S, U2)


# speedup vs baseline: 3.1299x; 1.0682x over previous
import functools

import jax
import jax.numpy as jnp
from jax import lax
from jax.experimental import pallas as pl
from jax.experimental.pallas import tpu as pltpu
from jax.experimental.pallas import tpu_sc as plsc

_INTERPRET = False

_E = 8
_K = 2
_TM = 256


def _erf(x):
    return jax.lax.erf(x)


def _gelu(x):
    return 0.5 * x * (1.0 + _erf(x * 0.7071067811865476))


def _router_body(x_ref, wg_ref, bg_ref, idx_ref, w_ref):
    x = x_ref[...]
    wg = wg_ref[...]
    logits = jnp.dot(x, wg, preferred_element_type=jnp.float32) + bg_ref[...][None, :]
    m = jnp.max(logits, axis=-1, keepdims=True)
    p = jnp.exp(logits - m)
    g = p / jnp.sum(p, axis=-1, keepdims=True)
    lane = jax.lax.broadcasted_iota(jnp.int32, g.shape, 1)
    v0 = jnp.max(g, axis=-1, keepdims=True)
    e0 = jnp.min(jnp.where(g == v0, lane, _E), axis=-1, keepdims=True)
    g2 = jnp.where(lane == e0, -1.0, g)
    v1 = jnp.max(g2, axis=-1, keepdims=True)
    e1 = jnp.min(jnp.where(g2 == v1, lane, _E), axis=-1, keepdims=True)
    s = v0 + v1 + 1e-9
    idx_ref[...] = jnp.concatenate([e0, e1], axis=1)
    w_ref[...] = jnp.concatenate([v0 / s, v1 / s], axis=1)


def _router(x2d, Wg, bg):
    T, D = x2d.shape
    TMR = 512
    return pl.pallas_call(
        _router_body,
        grid=(T // TMR,),
        in_specs=[
            pl.BlockSpec((TMR, D), lambda i: (i, 0)),
            pl.BlockSpec((D, _E), lambda i: (0, 0)),
            pl.BlockSpec((_E,), lambda i: (0,)),
        ],
        out_specs=[
            pl.BlockSpec((TMR, _K), lambda i: (i, 0)),
            pl.BlockSpec((TMR, _K), lambda i: (i, 0)),
        ],
        out_shape=[
            jax.ShapeDtypeStruct((T, _K), jnp.int32),
            jax.ShapeDtypeStruct((T, _K), jnp.float32),
        ],
        interpret=_INTERPRET,
    )(x2d, Wg, bg)


def _gmm_body(te_ref, xs_ref, w1_ref, b1_ref, w2_ref, b2_ref, rw_ref, out_ref,
              w1b_ref, w2b_ref):
    i = pl.program_id(0)
    new_w = jnp.logical_or(i == 0, te_ref[i] != te_ref[jnp.maximum(i - 1, 0)])

    @pl.when(new_w)
    def _():
        w1b_ref[...] = w1_ref[0].astype(jnp.bfloat16)
        w2b_ref[...] = w2_ref[0].astype(jnp.bfloat16)

    x = xs_ref[...]
    h = jnp.dot(x, w1b_ref[...], preferred_element_type=jnp.float32) + b1_ref[0, 0][None, :]
    h = _gelu(h)
    y = jnp.dot(h.astype(jnp.bfloat16), w2b_ref[...], preferred_element_type=jnp.float32)
    y = _gelu(y + b2_ref[0, 0][None, :])
    out_ref[...] = y * rw_ref[0, 0][:, None]


def _gmm(tile_expert, xs, W1b, b1, W2b, b2, row_w3):
    CAP, D = xs.shape
    U1 = W1b.shape[2]
    U2 = W2b.shape[2]
    NT = CAP // _TM
    grid_spec = pltpu.PrefetchScalarGridSpec(
        num_scalar_prefetch=1,
        grid=(NT,),
        in_specs=[
            pl.BlockSpec((_TM, D), lambda i, te: (i, 0)),
            pl.BlockSpec((1, D, U1), lambda i, te: (te[i], 0, 0)),
            pl.BlockSpec((1, 1, U1), lambda i, te: (te[i], 0, 0)),
            pl.BlockSpec((1, U1, U2), lambda i, te: (te[i], 0, 0)),
            pl.BlockSpec((1, 1, U2), lambda i, te: (te[i], 0, 0)),
            pl.BlockSpec((1, 1, _TM), lambda i, te: (i, 0, 0)),
        ],
        out_specs=pl.BlockSpec((_TM, U2), lambda i, te: (i, 0)),
        scratch_shapes=[
            pltpu.VMEM((D, U1), jnp.bfloat16),
            pltpu.VMEM((U1, U2), jnp.bfloat16),
        ],
    )
    return pl.pallas_call(
        _gmm_body,
        grid_spec=grid_spec,
        out_shape=jax.ShapeDtypeStruct((CAP, U2), jnp.float32),
        interpret=_INTERPRET,
    )(tile_expert, xs, W1b, b1, W2b, b2, row_w3)


def _combine(ysw, pos_flat, T, U2):
    NW = 32
    per = T // NW
    CT = 16
    nch = per // CT
    mesh = plsc.VectorSubcoreMesh(core_axis_name="c", subcore_axis_name="s")

    @functools.partial(
        pl.kernel,
        mesh=mesh,
        out_type=jax.ShapeDtypeStruct((T, U2), jnp.float32),
        scratch_types=[
            pltpu.VMEM((nch, 2 * CT), jnp.int32),
            pltpu.VMEM((2 * CT, U2), jnp.float32),
            pltpu.VMEM((2 * CT, U2), jnp.float32),
            pltpu.VMEM((CT, U2), jnp.float32),
            pltpu.SemaphoreType.DMA,
            pltpu.SemaphoreType.DMA,
        ],
    )
    def k(ysw_hbm, pos_hbm, out_hbm, idx_v, rows_a, rows_b, out_v, sem_a, sem_b):
        wid = lax.axis_index("s") * 2 + lax.axis_index("c")
        tok0 = wid * per
        pltpu.sync_copy(pos_hbm.at[wid], idx_v)
        bufs = [(rows_a, sem_a), (rows_b, sem_b)]
        handles = [None] * nch
        handles[0] = pltpu.async_copy(ysw_hbm.at[idx_v.at[0]], rows_a, sem_a)
        for c in range(nch):
            rows_v, _ = bufs[c % 2]
            nrows, nsem = bufs[(c + 1) % 2]
            if c + 1 < nch:
                handles[c + 1] = pltpu.async_copy(
                    ysw_hbm.at[idx_v.at[c + 1]], nrows, nsem)
            handles[c].wait()

            def tok(j, _):
                def col(cc, _):
                    for u in range(4):
                        sl = pl.ds(cc * 64 + u * 16, 16)
                        out_v[j, sl] = rows_v[2 * j, sl] + rows_v[2 * j + 1, sl]
                    return 0
                lax.fori_loop(0, U2 // 64, col, 0)
                return 0
            lax.fori_loop(0, CT, tok, 0)
            pltpu.sync_copy(out_v, out_hbm.at[pl.ds(tok0 + c * CT, CT)])

    return k(ysw, pos_flat.reshape(NW, nch, 2 * CT))


def kernel(inputs, Wg, bg, W1, b1, W2, b2):
    B, S, D = inputs.shape
    T = B * S
    U1 = W1.shape[2]
    U2 = W2.shape[2]
    x2d = inputs.reshape(T, D)

    idx_pair, w_pair = _router(x2d, Wg, bg)

    lane = jnp.arange(_E, dtype=jnp.int32)
    expert_flat = idx_pair.reshape(-1)

    match_t = (lane[:, None] == expert_flat[None, :]).astype(jnp.int32)
    ranks_t = jnp.cumsum(match_t, axis=1)
    rank = jnp.sum(match_t * ranks_t, axis=0) - 1
    counts = ranks_t[:, -1]
    padded = ((counts + _TM - 1) // _TM) * _TM
    offs = jnp.concatenate([jnp.zeros((1,), jnp.int32),
                            jnp.cumsum(padded)[:-1].astype(jnp.int32)])
    pos = offs[expert_flat] + rank

    CAP = _K * T + _E * _TM
    NT = CAP // _TM
    ends = (offs + padded) // _TM
    tiles = jnp.arange(NT, dtype=jnp.int32)
    tile_expert = jnp.minimum(
        jnp.sum(tiles[:, None] >= ends[None, :], axis=1), _E - 1
    ).astype(jnp.int32)

    tok_ids = jnp.arange(_K * T, dtype=jnp.int32) // _K
    _, sorted_tok, sorted_w = lax.sort(
        (expert_flat, tok_ids, w_pair.reshape(-1)), num_keys=1, is_stable=True)
    offs_u = jnp.concatenate([jnp.zeros((1,), jnp.int32),
                              jnp.cumsum(counts)[:-1].astype(jnp.int32)])
    er = jnp.repeat(tile_expert, _TM)
    r = jnp.arange(CAP, dtype=jnp.int32)
    rank_in_e = r - offs[er]
    valid = rank_in_e < counts[er]
    p = jnp.clip(offs_u[er] + rank_in_e, 0, _K * T - 1)
    row_token = jnp.where(valid, sorted_tok[p], 0)
    row_w = jnp.where(valid, sorted_w[p], 0.0)

    xs = x2d.astype(jnp.bfloat16)[row_token]

    ysw = _gmm(tile_expert, xs, W1, b1.reshape(_E, 1, U1),
               W2, b2.reshape(_E, 1, U2),
               row_w.reshape(NT, 1, _TM))

    out = _combine(ysw, pos, T, U2)
    return out.reshape(B, S, U2)


# speedup vs baseline: 3.1370x; 1.0023x over previous
import functools

import jax
import jax.numpy as jnp
from jax import lax
from jax.experimental import pallas as pl
from jax.experimental.pallas import tpu as pltpu
from jax.experimental.pallas import tpu_sc as plsc

_INTERPRET = False

_E = 8
_K = 2
_TM = 128


def _erf(x):
    return jax.lax.erf(x)


def _gelu(x):
    return 0.5 * x * (1.0 + _erf(x * 0.7071067811865476))


def _router_body(x_ref, wg_ref, bg_ref, idx_ref, w_ref):
    x = x_ref[...]
    wg = wg_ref[...]
    logits = jnp.dot(x, wg, preferred_element_type=jnp.float32) + bg_ref[...][None, :]
    m = jnp.max(logits, axis=-1, keepdims=True)
    p = jnp.exp(logits - m)
    g = p / jnp.sum(p, axis=-1, keepdims=True)
    lane = jax.lax.broadcasted_iota(jnp.int32, g.shape, 1)
    v0 = jnp.max(g, axis=-1, keepdims=True)
    e0 = jnp.min(jnp.where(g == v0, lane, _E), axis=-1, keepdims=True)
    g2 = jnp.where(lane == e0, -1.0, g)
    v1 = jnp.max(g2, axis=-1, keepdims=True)
    e1 = jnp.min(jnp.where(g2 == v1, lane, _E), axis=-1, keepdims=True)
    s = v0 + v1 + 1e-9
    idx_ref[...] = jnp.concatenate([e0, e1], axis=1)
    w_ref[...] = jnp.concatenate([v0 / s, v1 / s], axis=1)


def _router(x2d, Wg, bg):
    T, D = x2d.shape
    TMR = 512
    return pl.pallas_call(
        _router_body,
        grid=(T // TMR,),
        in_specs=[
            pl.BlockSpec((TMR, D), lambda i: (i, 0)),
            pl.BlockSpec((D, _E), lambda i: (0, 0)),
            pl.BlockSpec((_E,), lambda i: (0,)),
        ],
        out_specs=[
            pl.BlockSpec((TMR, _K), lambda i: (i, 0)),
            pl.BlockSpec((TMR, _K), lambda i: (i, 0)),
        ],
        out_shape=[
            jax.ShapeDtypeStruct((T, _K), jnp.int32),
            jax.ShapeDtypeStruct((T, _K), jnp.float32),
        ],
        interpret=_INTERPRET,
    )(x2d, Wg, bg)


def _gmm_body(te_ref, xs_ref, w1_ref, b1_ref, w2_ref, b2_ref, rw_ref, out_ref,
              w1b_ref, w2b_ref):
    i = pl.program_id(0)
    new_w = jnp.logical_or(i == 0, te_ref[i] != te_ref[jnp.maximum(i - 1, 0)])

    @pl.when(new_w)
    def _():
        w1b_ref[...] = w1_ref[0].astype(jnp.bfloat16)
        w2b_ref[...] = w2_ref[0].astype(jnp.bfloat16)

    x = xs_ref[...]
    h = jnp.dot(x, w1b_ref[...], preferred_element_type=jnp.float32) + b1_ref[0, 0][None, :]
    h = _gelu(h)
    y = jnp.dot(h.astype(jnp.bfloat16), w2b_ref[...], preferred_element_type=jnp.float32)
    y = _gelu(y + b2_ref[0, 0][None, :])
    out_ref[...] = y * rw_ref[0, 0][:, None]


def _gmm(tile_expert, xs, W1b, b1, W2b, b2, row_w3):
    CAP, D = xs.shape
    U1 = W1b.shape[2]
    U2 = W2b.shape[2]
    NT = CAP // _TM
    grid_spec = pltpu.PrefetchScalarGridSpec(
        num_scalar_prefetch=1,
        grid=(NT,),
        in_specs=[
            pl.BlockSpec((_TM, D), lambda i, te: (i, 0)),
            pl.BlockSpec((1, D, U1), lambda i, te: (te[i], 0, 0)),
            pl.BlockSpec((1, 1, U1), lambda i, te: (te[i], 0, 0)),
            pl.BlockSpec((1, U1, U2), lambda i, te: (te[i], 0, 0)),
            pl.BlockSpec((1, 1, U2), lambda i, te: (te[i], 0, 0)),
            pl.BlockSpec((1, 1, _TM), lambda i, te: (i, 0, 0)),
        ],
        out_specs=pl.BlockSpec((_TM, U2), lambda i, te: (i, 0)),
        scratch_shapes=[
            pltpu.VMEM((D, U1), jnp.bfloat16),
            pltpu.VMEM((U1, U2), jnp.bfloat16),
        ],
    )
    return pl.pallas_call(
        _gmm_body,
        grid_spec=grid_spec,
        out_shape=jax.ShapeDtypeStruct((CAP, U2), jnp.float32),
        interpret=_INTERPRET,
    )(tile_expert, xs, W1b, b1, W2b, b2, row_w3)


def _combine(ysw, pos_flat, T, U2):
    NW = 32
    per = T // NW
    CT = 16
    nch = per // CT
    mesh = plsc.VectorSubcoreMesh(core_axis_name="c", subcore_axis_name="s")

    @functools.partial(
        pl.kernel,
        mesh=mesh,
        out_type=jax.ShapeDtypeStruct((T, U2), jnp.float32),
        scratch_types=[
            pltpu.VMEM((nch, 2 * CT), jnp.int32),
            pltpu.VMEM((2 * CT, U2), jnp.float32),
            pltpu.VMEM((2 * CT, U2), jnp.float32),
            pltpu.VMEM((CT, U2), jnp.float32),
            pltpu.SemaphoreType.DMA,
            pltpu.SemaphoreType.DMA,
        ],
    )
    def k(ysw_hbm, pos_hbm, out_hbm, idx_v, rows_a, rows_b, out_v, sem_a, sem_b):
        wid = lax.axis_index("s") * 2 + lax.axis_index("c")
        tok0 = wid * per
        pltpu.sync_copy(pos_hbm.at[wid], idx_v)
        bufs = [(rows_a, sem_a), (rows_b, sem_b)]
        handles = [None] * nch
        handles[0] = pltpu.async_copy(ysw_hbm.at[idx_v.at[0]], rows_a, sem_a)
        for c in range(nch):
            rows_v, _ = bufs[c % 2]
            nrows, nsem = bufs[(c + 1) % 2]
            if c + 1 < nch:
                handles[c + 1] = pltpu.async_copy(
                    ysw_hbm.at[idx_v.at[c + 1]], nrows, nsem)
            handles[c].wait()

            def tok(j, _):
                def col(cc, _):
                    for u in range(4):
                        sl = pl.ds(cc * 64 + u * 16, 16)
                        out_v[j, sl] = rows_v[2 * j, sl] + rows_v[2 * j + 1, sl]
                    return 0
                lax.fori_loop(0, U2 // 64, col, 0)
                return 0
            lax.fori_loop(0, CT, tok, 0)
            pltpu.sync_copy(out_v, out_hbm.at[pl.ds(tok0 + c * CT, CT)])

    return k(ysw, pos_flat.reshape(NW, nch, 2 * CT))


def kernel(inputs, Wg, bg, W1, b1, W2, b2):
    B, S, D = inputs.shape
    T = B * S
    U1 = W1.shape[2]
    U2 = W2.shape[2]
    x2d = inputs.reshape(T, D)

    idx_pair, w_pair = _router(x2d, Wg, bg)

    lane = jnp.arange(_E, dtype=jnp.int32)
    expert_flat = idx_pair.reshape(-1)

    match_t = (lane[:, None] == expert_flat[None, :]).astype(jnp.int32)
    ranks_t = jnp.cumsum(match_t, axis=1)
    rank = jnp.sum(match_t * ranks_t, axis=0) - 1
    counts = ranks_t[:, -1]
    padded = ((counts + _TM - 1) // _TM) * _TM
    offs = jnp.concatenate([jnp.zeros((1,), jnp.int32),
                            jnp.cumsum(padded)[:-1].astype(jnp.int32)])
    pos = offs[expert_flat] + rank

    CAP = _K * T + _E * _TM
    NT = CAP // _TM
    ends = (offs + padded) // _TM
    tiles = jnp.arange(NT, dtype=jnp.int32)
    tile_expert = jnp.minimum(
        jnp.sum(tiles[:, None] >= ends[None, :], axis=1), _E - 1
    ).astype(jnp.int32)

    tok_ids = jnp.arange(_K * T, dtype=jnp.int32) // _K
    _, sorted_tok, sorted_w = lax.sort(
        (expert_flat, tok_ids, w_pair.reshape(-1)), num_keys=1, is_stable=True)
    offs_u = jnp.concatenate([jnp.zeros((1,), jnp.int32),
                              jnp.cumsum(counts)[:-1].astype(jnp.int32)])
    er = jnp.repeat(tile_expert, _TM)
    r = jnp.arange(CAP, dtype=jnp.int32)
    rank_in_e = r - offs[er]
    valid = rank_in_e < counts[er]
    p = jnp.clip(offs_u[er] + rank_in_e, 0, _K * T - 1)
    row_token = jnp.where(valid, sorted_tok[p], 0)
    row_w = jnp.where(valid, sorted_w[p], 0.0)

    xs = x2d.astype(jnp.bfloat16)[row_token]

    ysw = _gmm(tile_expert, xs, W1, b1.reshape(_E, 1, U1),
               W2, b2.reshape(_E, 1, U2),
               row_w.reshape(NT, 1, _TM))

    out = _combine(ysw, pos, T, U2)
    return out.reshape(B, S, U2)


# speedup vs baseline: 3.1982x; 1.0195x over previous
import functools

import jax
import jax.numpy as jnp
from jax import lax
from jax.experimental import pallas as pl
from jax.experimental.pallas import tpu as pltpu
from jax.experimental.pallas import tpu_sc as plsc

_INTERPRET = False

_E = 8
_K = 2
_TM = 128


def _erf(x):
    return jax.lax.erf(x)


def _gelu(x):
    return 0.5 * x * (1.0 + _erf(x * 0.7071067811865476))


def _router_body(x_ref, wg_ref, bg_ref, idx_ref, w_ref, xbf_ref):
    x = x_ref[...]
    wg = wg_ref[...]
    logits = jnp.dot(x, wg, preferred_element_type=jnp.float32) + bg_ref[...][None, :]
    m = jnp.max(logits, axis=-1, keepdims=True)
    p = jnp.exp(logits - m)
    g = p / jnp.sum(p, axis=-1, keepdims=True)
    lane = jax.lax.broadcasted_iota(jnp.int32, g.shape, 1)
    v0 = jnp.max(g, axis=-1, keepdims=True)
    e0 = jnp.min(jnp.where(g == v0, lane, _E), axis=-1, keepdims=True)
    g2 = jnp.where(lane == e0, -1.0, g)
    v1 = jnp.max(g2, axis=-1, keepdims=True)
    e1 = jnp.min(jnp.where(g2 == v1, lane, _E), axis=-1, keepdims=True)
    s = v0 + v1 + 1e-9
    idx_ref[...] = jnp.concatenate([e0, e1], axis=1)
    w_ref[...] = jnp.concatenate([v0 / s, v1 / s], axis=1)
    xbf_ref[...] = x.astype(jnp.bfloat16)


def _router(x2d, Wg, bg):
    T, D = x2d.shape
    TMR = 512
    return pl.pallas_call(
        _router_body,
        grid=(T // TMR,),
        in_specs=[
            pl.BlockSpec((TMR, D), lambda i: (i, 0)),
            pl.BlockSpec((D, _E), lambda i: (0, 0)),
            pl.BlockSpec((_E,), lambda i: (0,)),
        ],
        out_specs=[
            pl.BlockSpec((TMR, _K), lambda i: (i, 0)),
            pl.BlockSpec((TMR, _K), lambda i: (i, 0)),
            pl.BlockSpec((TMR, D), lambda i: (i, 0)),
        ],
        out_shape=[
            jax.ShapeDtypeStruct((T, _K), jnp.int32),
            jax.ShapeDtypeStruct((T, _K), jnp.float32),
            jax.ShapeDtypeStruct((T, D), jnp.bfloat16),
        ],
        interpret=_INTERPRET,
    )(x2d, Wg, bg)


def _gmm_body(te_ref, xs_ref, w1_ref, b1_ref, w2_ref, b2_ref, rw_ref, out_ref,
              w1b_ref, w2b_ref):
    i = pl.program_id(0)
    new_w = jnp.logical_or(i == 0, te_ref[i] != te_ref[jnp.maximum(i - 1, 0)])

    @pl.when(new_w)
    def _():
        w1b_ref[...] = w1_ref[0].astype(jnp.bfloat16)
        w2b_ref[...] = w2_ref[0].astype(jnp.bfloat16)

    x = xs_ref[...]
    h = jnp.dot(x, w1b_ref[...], preferred_element_type=jnp.float32) + b1_ref[0, 0][None, :]
    h = _gelu(h)
    y = jnp.dot(h.astype(jnp.bfloat16), w2b_ref[...], preferred_element_type=jnp.float32)
    y = _gelu(y + b2_ref[0, 0][None, :])
    out_ref[...] = y * rw_ref[0, 0][:, None]


def _gmm(tile_expert, xs, W1b, b1, W2b, b2, row_w3):
    CAP, D = xs.shape
    U1 = W1b.shape[2]
    U2 = W2b.shape[2]
    NT = CAP // _TM
    grid_spec = pltpu.PrefetchScalarGridSpec(
        num_scalar_prefetch=1,
        grid=(NT,),
        in_specs=[
            pl.BlockSpec((_TM, D), lambda i, te: (i, 0)),
            pl.BlockSpec((1, D, U1), lambda i, te: (te[i], 0, 0)),
            pl.BlockSpec((1, 1, U1), lambda i, te: (te[i], 0, 0)),
            pl.BlockSpec((1, U1, U2), lambda i, te: (te[i], 0, 0)),
            pl.BlockSpec((1, 1, U2), lambda i, te: (te[i], 0, 0)),
            pl.BlockSpec((1, 1, _TM), lambda i, te: (i, 0, 0)),
        ],
        out_specs=pl.BlockSpec((_TM, U2), lambda i, te: (i, 0)),
        scratch_shapes=[
            pltpu.VMEM((D, U1), jnp.bfloat16),
            pltpu.VMEM((U1, U2), jnp.bfloat16),
        ],
    )
    return pl.pallas_call(
        _gmm_body,
        grid_spec=grid_spec,
        out_shape=jax.ShapeDtypeStruct((CAP, U2), jnp.float32),
        interpret=_INTERPRET,
    )(tile_expert, xs, W1b, b1, W2b, b2, row_w3)


def _combine(ysw, pos_flat, T, U2):
    NW = 32
    per = T // NW
    CT = 16
    nch = per // CT
    mesh = plsc.VectorSubcoreMesh(core_axis_name="c", subcore_axis_name="s")

    @functools.partial(
        pl.kernel,
        mesh=mesh,
        out_type=jax.ShapeDtypeStruct((T, U2), jnp.float32),
        scratch_types=[
            pltpu.VMEM((nch, 2 * CT), jnp.int32),
            pltpu.VMEM((2 * CT, U2), jnp.float32),
            pltpu.VMEM((2 * CT, U2), jnp.float32),
            pltpu.VMEM((CT, U2), jnp.float32),
            pltpu.SemaphoreType.DMA,
            pltpu.SemaphoreType.DMA,
        ],
    )
    def k(ysw_hbm, pos_hbm, out_hbm, idx_v, rows_a, rows_b, out_v, sem_a, sem_b):
        wid = lax.axis_index("s") * 2 + lax.axis_index("c")
        tok0 = wid * per
        pltpu.sync_copy(pos_hbm.at[wid], idx_v)
        bufs = [(rows_a, sem_a), (rows_b, sem_b)]
        handles = [None] * nch
        handles[0] = pltpu.async_copy(ysw_hbm.at[idx_v.at[0]], rows_a, sem_a)
        for c in range(nch):
            rows_v, _ = bufs[c % 2]
            nrows, nsem = bufs[(c + 1) % 2]
            if c + 1 < nch:
                handles[c + 1] = pltpu.async_copy(
                    ysw_hbm.at[idx_v.at[c + 1]], nrows, nsem)
            handles[c].wait()

            def tok(j, _):
                def col(cc, _):
                    for u in range(4):
                        sl = pl.ds(cc * 64 + u * 16, 16)
                        out_v[j, sl] = rows_v[2 * j, sl] + rows_v[2 * j + 1, sl]
                    return 0
                lax.fori_loop(0, U2 // 64, col, 0)
                return 0
            lax.fori_loop(0, CT, tok, 0)
            pltpu.sync_copy(out_v, out_hbm.at[pl.ds(tok0 + c * CT, CT)])

    return k(ysw, pos_flat.reshape(NW, nch, 2 * CT))


def kernel(inputs, Wg, bg, W1, b1, W2, b2):
    B, S, D = inputs.shape
    T = B * S
    U1 = W1.shape[2]
    U2 = W2.shape[2]
    x2d = inputs.reshape(T, D)

    idx_pair, w_pair, x_bf = _router(x2d, Wg, bg)

    lane = jnp.arange(_E, dtype=jnp.int32)
    expert_flat = idx_pair.reshape(-1)

    match_t = (lane[:, None] == expert_flat[None, :]).astype(jnp.int32)
    ranks_t = jnp.cumsum(match_t, axis=1)
    rank = jnp.sum(match_t * ranks_t, axis=0) - 1
    counts = ranks_t[:, -1]
    padded = ((counts + _TM - 1) // _TM) * _TM
    offs = jnp.concatenate([jnp.zeros((1,), jnp.int32),
                            jnp.cumsum(padded)[:-1].astype(jnp.int32)])
    pos = offs[expert_flat] + rank

    CAP = _K * T + _E * _TM
    NT = CAP // _TM
    ends = (offs + padded) // _TM
    tiles = jnp.arange(NT, dtype=jnp.int32)
    tile_expert = jnp.minimum(
        jnp.sum(tiles[:, None] >= ends[None, :], axis=1), _E - 1
    ).astype(jnp.int32)

    tok_ids = jnp.arange(_K * T, dtype=jnp.int32) // _K
    _, sorted_tok, sorted_w = lax.sort(
        (expert_flat, tok_ids, w_pair.reshape(-1)), num_keys=1, is_stable=True)
    offs_u = jnp.concatenate([jnp.zeros((1,), jnp.int32),
                              jnp.cumsum(counts)[:-1].astype(jnp.int32)])
    er = jnp.repeat(tile_expert, _TM)
    r = jnp.arange(CAP, dtype=jnp.int32)
    rank_in_e = r - offs[er]
    valid = rank_in_e < counts[er]
    p = jnp.clip(offs_u[er] + rank_in_e, 0, _K * T - 1)
    row_token = jnp.where(valid, sorted_tok[p], 0)
    row_w = jnp.where(valid, sorted_w[p], 0.0)

    xs = x_bf[row_token]

    ysw = _gmm(tile_expert, xs, W1, b1.reshape(_E, 1, U1),
               W2, b2.reshape(_E, 1, U2),
               row_w.reshape(NT, 1, _TM))

    out = _combine(ysw, pos, T, U2)
    return out.reshape(B, S, U2)
